# baseline (speedup 1.0000x reference)
import functools

import jax
import jax.numpy as jnp
from jax import lax
from jax.experimental import pallas as pl
from jax.experimental.pallas import tpu as pltpu
from jax.experimental.pallas import tpu_sc as plsc

N = 10000
E = 160000
D = 256
DH = 128
NS = 16
C = 128
RPT = 80
E_PAD = NS * RPT * C
N2 = 10240
N_TILE = N2 // NS
DUMP = N2 - 1
R = 256
GRID = (N + R - 1) // R


def _sc_agg(p0, p1, src2d, dst2d, zrows, zcnt, ones, with_cnt):
    out_types = [jax.ShapeDtypeStruct((N2, DH), jnp.float32),
                 jax.ShapeDtypeStruct((N2, DH), jnp.float32)]
    if with_cnt:
        out_types.append(jax.ShapeDtypeStruct((N2,), jnp.float32))
    scratch = [
        pltpu.VMEM_SHARED((N2, DH), jnp.float32),
        pltpu.VMEM_SHARED((N2,), jnp.float32),
        pltpu.VMEM((RPT // 2, C), jnp.int32),
        pltpu.VMEM((RPT // 2, C), jnp.int32),
        pltpu.VMEM((C, DH), jnp.float32),
        pltpu.VMEM((C, DH), jnp.float32),
        pltpu.VMEM((C,), jnp.float32),
        pltpu.SemaphoreType.DMA,
        pltpu.SemaphoreType.DMA,
    ]
    mesh = plsc.VectorSubcoreMesh(core_axis_name="c", subcore_axis_name="s")

    def body(p0_hbm, p1_hbm, src_hbm, dst_hbm, zr_hbm, zc_hbm, on_hbm,
             *rest):
        if with_cnt:
            agg0_hbm, agg1_hbm, cnt_hbm = rest[:3]
            rest = rest[3:]
        else:
            agg0_hbm, agg1_hbm = rest[:2]
            cnt_hbm = None
            rest = rest[2:]
        acc_sp, cnt_sp, sidx_v, didx_v, rows0_v, rows1_v, ones_v, \
            sem0, sem1 = rest
        c = lax.axis_index("c")
        s = lax.axis_index("s")
        sl = pl.ds(s * N_TILE, N_TILE)

        pltpu.sync_copy(zr_hbm, acc_sp.at[sl])
        if with_cnt:
            @pl.when(jnp.logical_and(c == 0, s == 0))
            def _():
                pltpu.sync_copy(zc_hbm, cnt_sp)
            pltpu.sync_copy(on_hbm, ones_v)
        plsc.subcore_barrier()

        HRPT = RPT // 2

        def run_edges(p_hbm, do_cnt):
            def fire(j, rows_v, sem):
                pltpu.async_copy(p_hbm.at[sidx_v.at[j]], rows_v, sem)

            def drain(j, rows_v, sem):
                pltpu.make_async_copy(p_hbm.at[sidx_v.at[j]], rows_v,
                                      sem).wait()

            def scat(j, rows_v):
                return
                pltpu.sync_copy(rows_v, acc_sp.at[didx_v.at[j]], add=True)
                if do_cnt:
                    pltpu.sync_copy(ones_v, cnt_sp.at[didx_v.at[j]],
                                    add=True)

            for half in range(2):
                rsl = pl.ds(s * RPT + half * HRPT, HRPT)
                pltpu.sync_copy(src_hbm.at[rsl], sidx_v)
                pltpu.sync_copy(dst_hbm.at[rsl], didx_v)
                fire(0, rows0_v, sem0)

                def step(k, carry):
                    j0 = 2 * k
                    fire(j0 + 1, rows1_v, sem1)
                    drain(j0, rows0_v, sem0)
                    scat(j0, rows0_v)

                    @pl.when(k < HRPT // 2 - 1)
                    def _():
                        fire(j0 + 2, rows0_v, sem0)
                    drain(j0 + 1, rows1_v, sem1)
                    scat(j0 + 1, rows1_v)
                    return carry

                lax.fori_loop(0, HRPT // 2, step, 0)

        @pl.when(c == 0)
        def _():
            run_edges(p0_hbm, with_cnt)

        @pl.when(c == 1)
        def _():
            run_edges(p1_hbm, False)

        plsc.subcore_barrier()

        @pl.when(c == 0)
        def _():
            pltpu.sync_copy(acc_sp.at[sl], agg0_hbm.at[sl])
            if with_cnt:
                csl = pl.ds(s * N_TILE, N_TILE)
                pltpu.sync_copy(cnt_sp.at[csl], cnt_hbm.at[csl])

        @pl.when(c == 1)
        def _():
            pltpu.sync_copy(acc_sp.at[sl], agg1_hbm.at[sl])

    fn = pl.kernel(body, mesh=mesh, out_type=out_types,
                   scratch_types=scratch)
    return fn(p0, p1, src2d, dst2d, zrows, zcnt, ones)


def _row_spec(w):
    return pl.BlockSpec((R, w), lambda i: (i, 0))


def _full_spec(shape):
    return pl.BlockSpec(shape, lambda i: (0,) * len(shape))


def _enc_body(x_ref, we_ref, be_ref, wl_ref, h_ref, p0_ref, p1_ref):
    h = jnp.maximum(
        jnp.dot(x_ref[...], we_ref[...],
                preferred_element_type=jnp.float32) + be_ref[...], 0.0)
    p = jnp.dot(h, wl_ref[...], preferred_element_type=jnp.float32)
    h_ref[...] = h
    p0_ref[...] = p[:, :DH]
    p1_ref[...] = p[:, DH:]


def _enc_call(x, W_enc, b_enc2, Wl0):
    return pl.pallas_call(
        _enc_body,
        grid=(GRID,),
        in_specs=[_row_spec(D), _full_spec((D, D)), _full_spec((1, D)),
                  _full_spec((D, D))],
        out_specs=[_row_spec(D), _row_spec(DH), _row_spec(DH)],
        out_shape=[jax.ShapeDtypeStruct((N, D), jnp.float32),
                   jax.ShapeDtypeStruct((N2, DH), jnp.float32),
                   jax.ShapeDtypeStruct((N2, DH), jnp.float32)],
    )(x, W_enc, b_enc2, Wl0)


def _post_common(a0_ref, a1_ref, cnt_ref, h_ref, wr_ref, bl_ref, g_ref,
                 be_ref):
    cnt = jnp.maximum(cnt_ref[...], 1.0)
    mean_wl = jnp.concatenate([a0_ref[...], a1_ref[...]], axis=1) / cnt
    h = h_ref[...]
    u = mean_wl + bl_ref[...] + jnp.dot(
        h, wr_ref[...], preferred_element_type=jnp.float32) + h
    r = jnp.maximum(u, 0.0)
    mu = jnp.mean(r, axis=1, keepdims=True)
    var = jnp.mean((r - mu) ** 2, axis=1, keepdims=True)
    return (r - mu) / jnp.sqrt(var + 1e-5) * g_ref[...] + be_ref[...]


def _mid_body(a0_ref, a1_ref, cnt_ref, h_ref, wr_ref, bl_ref, g_ref,
              be_ref, wln_ref, hn_ref, p0_ref, p1_ref):
    hn = _post_common(a0_ref, a1_ref, cnt_ref, h_ref, wr_ref, bl_ref,
                      g_ref, be_ref)
    hn_ref[...] = hn
    p = jnp.dot(hn, wln_ref[...], preferred_element_type=jnp.float32)
    p0_ref[...] = p[:, :DH]
    p1_ref[...] = p[:, DH:]


def _mid_call(a0, a1, cnt2d, h, Wr, bl2, g2, be2, Wl_next):
    return pl.pallas_call(
        _mid_body,
        grid=(GRID,),
        in_specs=[_row_spec(DH), _row_spec(DH), _row_spec(1), _row_spec(D),
                  _full_spec((D, D)), _full_spec((1, D)), _full_spec((1, D)),
                  _full_spec((1, D)), _full_spec((D, D))],
        out_specs=[_row_spec(D), _row_spec(DH), _row_spec(DH)],
        out_shape=[jax.ShapeDtypeStruct((N, D), jnp.float32),
                   jax.ShapeDtypeStruct((N2, DH), jnp.float32),
                   jax.ShapeDtypeStruct((N2, DH), jnp.float32)],
    )(a0, a1, cnt2d, h, Wr, bl2, g2, be2, Wl_next)


def _last_body(a0_ref, a1_ref, cnt_ref, h_ref, henc_ref, h1_ref, wr_ref,
               bl_ref, g_ref, be_ref, out_ref):
    hn = _post_common(a0_ref, a1_ref, cnt_ref, h_ref, wr_ref, bl_ref,
                      g_ref, be_ref)
    out_ref[...] = hn + henc_ref[...] + h1_ref[...]


def _last_call(a0, a1, cnt2d, h, h_enc, h1, Wr, bl2, g2, be2):
    return pl.pallas_call(
        _last_body,
        grid=(GRID,),
        in_specs=[_row_spec(DH), _row_spec(DH), _row_spec(1), _row_spec(D),
                  _row_spec(D), _row_spec(D), _full_spec((D, D)),
                  _full_spec((1, D)), _full_spec((1, D)), _full_spec((1, D))],
        out_specs=_row_spec(D),
        out_shape=jax.ShapeDtypeStruct((N, D), jnp.float32),
    )(a0, a1, cnt2d, h, h_enc, h1, Wr, bl2, g2, be2)


def kernel(x, edge_index, W_enc, b_enc,
           Wl0, bl0, Wr0, g0, be0,
           Wl1, bl1, Wr1, g1, be1,
           Wl2, bl2, Wr2, g2, be2):
    pad = E_PAD - E
    src1d = jnp.concatenate([edge_index[0], jnp.zeros((pad,), jnp.int32)])
    dst1d = jnp.concatenate([edge_index[1],
                             jnp.full((pad,), DUMP, jnp.int32)])
    src1d = src1d.reshape(E_PAD // C, C)
    dst1d = dst1d.reshape(E_PAD // C, C)
    zrows = jnp.zeros((N_TILE, DH), jnp.float32)
    zcnt = jnp.zeros((N2,), jnp.float32)
    ones = jnp.ones((C,), jnp.float32)

    r1 = lambda v: v.reshape(1, D)

    h_enc, p0, p1 = _enc_call(x, W_enc, r1(b_enc), Wl0)
    a0, a1, cntp = _sc_agg(p0, p1, src1d, dst1d, zrows, zcnt, ones, True)
    cnt2d = cntp[:N].reshape(N, 1)
    h1, q0, q1 = _mid_call(a0, a1, cnt2d, h_enc, Wr0, r1(bl0), r1(g0),
                           r1(be0), Wl1)
    b0, b1 = _sc_agg(q0, q1, src1d, dst1d, zrows, zcnt, ones, False)
    h2, t0, t1 = _mid_call(b0, b1, cnt2d, h1, Wr1, r1(bl1), r1(g1),
                           r1(be1), Wl2)
    c0, c1 = _sc_agg(t0, t1, src1d, dst1d, zrows, zcnt, ones, False)
    out = _last_call(c0, c1, cnt2d, h2, h_enc, h1, Wr2, r1(bl2), r1(g2),
                     r1(be2))
    return out


# speedup vs baseline: 1.0150x; 1.0150x over previous
import functools

import jax
import jax.numpy as jnp
from jax import lax
from jax.experimental import pallas as pl
from jax.experimental.pallas import tpu as pltpu
from jax.experimental.pallas import tpu_sc as plsc

N = 10000
E = 160000
D = 256
DH = 128
NS = 16
C = 64
RPT = 160
NBUF = 4
QC = 40
E_PAD = NS * RPT * C
N2 = 10240
N_TILE = N2 // NS
DUMP = N2 - 1
R = 256
GRID = (N + R - 1) // R


def _sc_agg(p0, p1, src2d, dst2d, zrows, zcnt, ones, with_cnt):
    out_types = [jax.ShapeDtypeStruct((N2, DH), jnp.float32),
                 jax.ShapeDtypeStruct((N2, DH), jnp.float32)]
    if with_cnt:
        out_types.append(jax.ShapeDtypeStruct((N2,), jnp.float32))
    scratch = [
        pltpu.VMEM_SHARED((N2, DH), jnp.float32),
        pltpu.VMEM_SHARED((N2,), jnp.float32),
        pltpu.VMEM((QC, C), jnp.int32),
        pltpu.VMEM((QC, C), jnp.int32),
    ] + [pltpu.VMEM((C, DH), jnp.float32) for _ in range(NBUF)] + [
        pltpu.VMEM((C,), jnp.float32),
    ] + [pltpu.SemaphoreType.DMA for _ in range(NBUF)]
    mesh = plsc.VectorSubcoreMesh(core_axis_name="c", subcore_axis_name="s")

    def body(p0_hbm, p1_hbm, src_hbm, dst_hbm, zr_hbm, zc_hbm, on_hbm,
             *rest):
        if with_cnt:
            agg0_hbm, agg1_hbm, cnt_hbm = rest[:3]
            rest = rest[3:]
        else:
            agg0_hbm, agg1_hbm = rest[:2]
            cnt_hbm = None
            rest = rest[2:]
        acc_sp, cnt_sp, sidx_v, didx_v = rest[:4]
        rows_bufs = rest[4:4 + NBUF]
        ones_v = rest[4 + NBUF]
        sems = rest[5 + NBUF:]
        c = lax.axis_index("c")
        s = lax.axis_index("s")
        sl = pl.ds(s * N_TILE, N_TILE)

        pltpu.sync_copy(zr_hbm, acc_sp.at[sl])
        if with_cnt:
            @pl.when(jnp.logical_and(c == 0, s == 0))
            def _():
                pltpu.sync_copy(zc_hbm, cnt_sp)
            pltpu.sync_copy(on_hbm, ones_v)
        plsc.subcore_barrier()

        def run_edges(p_hbm, do_cnt):
            def fire(j, b):
                pltpu.async_copy(p_hbm.at[sidx_v.at[j]], rows_bufs[b],
                                 sems[b])

            def drain(j, b):
                pltpu.make_async_copy(p_hbm.at[sidx_v.at[j]], rows_bufs[b],
                                      sems[b]).wait()

            def scat(j, b):
                pltpu.sync_copy(rows_bufs[b], acc_sp.at[didx_v.at[j]],
                                add=True)
                if do_cnt:
                    pltpu.sync_copy(ones_v, cnt_sp.at[didx_v.at[j]],
                                    add=True)

            for q in range(RPT // QC):
                rsl = pl.ds(s * RPT + q * QC, QC)
                pltpu.sync_copy(src_hbm.at[rsl], sidx_v)
                pltpu.sync_copy(dst_hbm.at[rsl], didx_v)
                for b in range(NBUF):
                    fire(b, b)

                def step(k, carry):
                    for b in range(NBUF):
                        j = NBUF * k + b
                        drain(j, b)
                        scat(j, b)

                        @pl.when(j + NBUF < QC)
                        def _():
                            fire(j + NBUF, b)
                    return carry

                lax.fori_loop(0, QC // NBUF, step, 0)

        @pl.when(c == 0)
        def _():
            run_edges(p0_hbm, with_cnt)

        @pl.when(c == 1)
        def _():
            run_edges(p1_hbm, False)

        plsc.subcore_barrier()

        @pl.when(c == 0)
        def _():
            pltpu.sync_copy(acc_sp.at[sl], agg0_hbm.at[sl])
            if with_cnt:
                csl = pl.ds(s * N_TILE, N_TILE)
                pltpu.sync_copy(cnt_sp.at[csl], cnt_hbm.at[csl])

        @pl.when(c == 1)
        def _():
            pltpu.sync_copy(acc_sp.at[sl], agg1_hbm.at[sl])

    fn = pl.kernel(body, mesh=mesh, out_type=out_types,
                   scratch_types=scratch)
    return fn(p0, p1, src2d, dst2d, zrows, zcnt, ones)


def _row_spec(w):
    return pl.BlockSpec((R, w), lambda i: (i, 0))


def _full_spec(shape):
    return pl.BlockSpec(shape, lambda i: (0,) * len(shape))


def _enc_body(x_ref, we_ref, be_ref, wl_ref, h_ref, p0_ref, p1_ref):
    h = jnp.maximum(
        jnp.dot(x_ref[...], we_ref[...],
                preferred_element_type=jnp.float32) + be_ref[...], 0.0)
    p = jnp.dot(h, wl_ref[...], preferred_element_type=jnp.float32)
    h_ref[...] = h
    p0_ref[...] = p[:, :DH]
    p1_ref[...] = p[:, DH:]


def _enc_call(x, W_enc, b_enc2, Wl0):
    return pl.pallas_call(
        _enc_body,
        grid=(GRID,),
        in_specs=[_row_spec(D), _full_spec((D, D)), _full_spec((1, D)),
                  _full_spec((D, D))],
        out_specs=[_row_spec(D), _row_spec(DH), _row_spec(DH)],
        out_shape=[jax.ShapeDtypeStruct((N, D), jnp.float32),
                   jax.ShapeDtypeStruct((N2, DH), jnp.float32),
                   jax.ShapeDtypeStruct((N2, DH), jnp.float32)],
    )(x, W_enc, b_enc2, Wl0)


def _post_common(a0_ref, a1_ref, cnt_ref, h_ref, wr_ref, bl_ref, g_ref,
                 be_ref):
    cnt = jnp.maximum(cnt_ref[...], 1.0)
    mean_wl = jnp.concatenate([a0_ref[...], a1_ref[...]], axis=1) / cnt
    h = h_ref[...]
    u = mean_wl + bl_ref[...] + jnp.dot(
        h, wr_ref[...], preferred_element_type=jnp.float32) + h
    r = jnp.maximum(u, 0.0)
    mu = jnp.mean(r, axis=1, keepdims=True)
    var = jnp.mean((r - mu) ** 2, axis=1, keepdims=True)
    return (r - mu) / jnp.sqrt(var + 1e-5) * g_ref[...] + be_ref[...]


def _mid_body(a0_ref, a1_ref, cnt_ref, h_ref, wr_ref, bl_ref, g_ref,
              be_ref, wln_ref, hn_ref, p0_ref, p1_ref):
    hn = _post_common(a0_ref, a1_ref, cnt_ref, h_ref, wr_ref, bl_ref,
                      g_ref, be_ref)
    hn_ref[...] = hn
    p = jnp.dot(hn, wln_ref[...], preferred_element_type=jnp.float32)
    p0_ref[...] = p[:, :DH]
    p1_ref[...] = p[:, DH:]


def _mid_call(a0, a1, cnt2d, h, Wr, bl2, g2, be2, Wl_next):
    return pl.pallas_call(
        _mid_body,
        grid=(GRID,),
        in_specs=[_row_spec(DH), _row_spec(DH), _row_spec(1), _row_spec(D),
                  _full_spec((D, D)), _full_spec((1, D)), _full_spec((1, D)),
                  _full_spec((1, D)), _full_spec((D, D))],
        out_specs=[_row_spec(D), _row_spec(DH), _row_spec(DH)],
        out_shape=[jax.ShapeDtypeStruct((N, D), jnp.float32),
                   jax.ShapeDtypeStruct((N2, DH), jnp.float32),
                   jax.ShapeDtypeStruct((N2, DH), jnp.float32)],
    )(a0, a1, cnt2d, h, Wr, bl2, g2, be2, Wl_next)


def _last_body(a0_ref, a1_ref, cnt_ref, h_ref, henc_ref, h1_ref, wr_ref,
               bl_ref, g_ref, be_ref, out_ref):
    hn = _post_common(a0_ref, a1_ref, cnt_ref, h_ref, wr_ref, bl_ref,
                      g_ref, be_ref)
    out_ref[...] = hn + henc_ref[...] + h1_ref[...]


def _last_call(a0, a1, cnt2d, h, h_enc, h1, Wr, bl2, g2, be2):
    return pl.pallas_call(
        _last_body,
        grid=(GRID,),
        in_specs=[_row_spec(DH), _row_spec(DH), _row_spec(1), _row_spec(D),
                  _row_spec(D), _row_spec(D), _full_spec((D, D)),
                  _full_spec((1, D)), _full_spec((1, D)), _full_spec((1, D))],
        out_specs=_row_spec(D),
        out_shape=jax.ShapeDtypeStruct((N, D), jnp.float32),
    )(a0, a1, cnt2d, h, h_enc, h1, Wr, bl2, g2, be2)


def kernel(x, edge_index, W_enc, b_enc,
           Wl0, bl0, Wr0, g0, be0,
           Wl1, bl1, Wr1, g1, be1,
           Wl2, bl2, Wr2, g2, be2):
    pad = E_PAD - E
    src1d = jnp.concatenate([edge_index[0], jnp.zeros((pad,), jnp.int32)])
    dst1d = jnp.concatenate([edge_index[1],
                             jnp.full((pad,), DUMP, jnp.int32)])
    src1d = src1d.reshape(E_PAD // C, C)
    dst1d = dst1d.reshape(E_PAD // C, C)
    zrows = jnp.zeros((N_TILE, DH), jnp.float32)
    zcnt = jnp.zeros((N2,), jnp.float32)
    ones = jnp.ones((C,), jnp.float32)

    r1 = lambda v: v.reshape(1, D)

    h_enc, p0, p1 = _enc_call(x, W_enc, r1(b_enc), Wl0)
    a0, a1, cntp = _sc_agg(p0, p1, src1d, dst1d, zrows, zcnt, ones, True)
    cnt2d = cntp[:N].reshape(N, 1)
    h1, q0, q1 = _mid_call(a0, a1, cnt2d, h_enc, Wr0, r1(bl0), r1(g0),
                           r1(be0), Wl1)
    b0, b1 = _sc_agg(q0, q1, src1d, dst1d, zrows, zcnt, ones, False)
    h2, t0, t1 = _mid_call(b0, b1, cnt2d, h1, Wr1, r1(bl1), r1(g1),
                           r1(be1), Wl2)
    c0, c1 = _sc_agg(t0, t1, src1d, dst1d, zrows, zcnt, ones, False)
    out = _last_call(c0, c1, cnt2d, h2, h_enc, h1, Wr2, r1(bl2), r1(g2),
                     r1(be2))
    return out


# speedup vs baseline: 1.4123x; 1.3914x over previous
import functools

import jax
import jax.numpy as jnp
from jax import lax
from jax.experimental import pallas as pl
from jax.experimental.pallas import tpu as pltpu
from jax.experimental.pallas import tpu_sc as plsc

N = 10000
E = 160000
D = 256
DQ = 64
NS = 16
C = 128
RPT = 80
QC = 40
E_PAD = NS * RPT * C
N2 = 10240
N_TILE = N2 // NS
DUMP = N2 - 1
R = 256
GRID = (N + R - 1) // R


def _sc_agg(pq, src2d, dst2d, zrows, zcnt, ones, with_cnt):
    out_types = [jax.ShapeDtypeStruct((N2, DQ), jnp.float32)
                 for _ in range(4)]
    if with_cnt:
        out_types.append(jax.ShapeDtypeStruct((N2,), jnp.float32))
    scratch = [
        pltpu.VMEM_SHARED((N2, DQ), jnp.float32),
        pltpu.VMEM_SHARED((N2, DQ), jnp.float32),
        pltpu.VMEM_SHARED((N2,), jnp.float32),
        pltpu.VMEM((QC, C), jnp.int32),
        pltpu.VMEM((QC, C), jnp.int32),
        pltpu.VMEM((C, DQ), jnp.float32),
        pltpu.VMEM((C, DQ), jnp.float32),
        pltpu.VMEM((C,), jnp.float32),
        pltpu.SemaphoreType.DMA,
        pltpu.SemaphoreType.DMA,
    ]
    mesh = plsc.VectorSubcoreMesh(core_axis_name="c", subcore_axis_name="s")

    def body(p0_hbm, p1_hbm, p2_hbm, p3_hbm, src_hbm, dst_hbm, zr_hbm,
             zc_hbm, on_hbm, *rest):
        if with_cnt:
            agg_hbm = rest[:4]
            cnt_hbm = rest[4]
            rest = rest[5:]
        else:
            agg_hbm = rest[:4]
            cnt_hbm = None
            rest = rest[4:]
        p_sp, acc_sp, cnt_sp, sidx_v, didx_v, rows0_v, rows1_v, ones_v, \
            sem0, sem1 = rest
        p_hbm = (p0_hbm, p1_hbm, p2_hbm, p3_hbm)
        c = lax.axis_index("c")
        s = lax.axis_index("s")
        sl = pl.ds(s * N_TILE, N_TILE)

        if with_cnt:
            @pl.when(jnp.logical_and(c == 0, s == 0))
            def _():
                pltpu.sync_copy(zc_hbm, cnt_sp)
            pltpu.sync_copy(on_hbm, ones_v)

        def run_edges(do_cnt):
            def fire(j, rows_v, sem):
                pltpu.async_copy(p_sp.at[sidx_v.at[j]], rows_v, sem)

            def drain(j, rows_v, sem):
                pltpu.make_async_copy(p_sp.at[sidx_v.at[j]], rows_v,
                                      sem).wait()

            def scat(j, rows_v):
                pltpu.sync_copy(rows_v, acc_sp.at[didx_v.at[j]], add=True)
                if do_cnt:
                    pltpu.sync_copy(ones_v, cnt_sp.at[didx_v.at[j]],
                                    add=True)

            for q in range(RPT // QC):
                rsl = pl.ds(s * RPT + q * QC, QC)
                pltpu.sync_copy(src_hbm.at[rsl], sidx_v)
                pltpu.sync_copy(dst_hbm.at[rsl], didx_v)
                fire(0, rows0_v, sem0)

                def step(k, carry):
                    j0 = 2 * k
                    fire(j0 + 1, rows1_v, sem1)
                    drain(j0, rows0_v, sem0)
                    scat(j0, rows0_v)

                    @pl.when(k < QC // 2 - 1)
                    def _():
                        fire(j0 + 2, rows0_v, sem0)
                    drain(j0 + 1, rows1_v, sem1)
                    scat(j0 + 1, rows1_v)
                    return carry

                lax.fori_loop(0, QC // 2, step, 0)

        for t in range(2):
            for cc in range(2):
                qn = 2 * cc + t

                @pl.when(c == cc)
                def _():
                    pltpu.sync_copy(p_hbm[qn].at[sl], p_sp.at[sl])
            pltpu.sync_copy(zr_hbm, acc_sp.at[sl])
            plsc.subcore_barrier()

            run_edges(with_cnt and t == 0)

            plsc.subcore_barrier()
            for cc in range(2):
                qn = 2 * cc + t

                @pl.when(c == cc)
                def _():
                    pltpu.sync_copy(acc_sp.at[sl], agg_hbm[qn].at[sl])
            if with_cnt and t == 0:
                @pl.when(c == 0)
                def _():
                    pltpu.sync_copy(cnt_sp.at[pl.ds(s * N_TILE, N_TILE)],
                                    cnt_hbm.at[pl.ds(s * N_TILE, N_TILE)])
            plsc.subcore_barrier()

    fn = pl.kernel(body, mesh=mesh, out_type=out_types,
                   scratch_types=scratch)
    return fn(*pq, src2d, dst2d, zrows, zcnt, ones)


def _row_spec(w):
    return pl.BlockSpec((R, w), lambda i: (i, 0))


def _full_spec(shape):
    return pl.BlockSpec(shape, lambda i: (0,) * len(shape))


def _q_outs():
    return ([_row_spec(DQ) for _ in range(4)],
            [jax.ShapeDtypeStruct((N2, DQ), jnp.float32) for _ in range(4)])


def _split_q(p, refs):
    for qn in range(4):
        refs[qn][...] = p[:, qn * DQ:(qn + 1) * DQ]


def _enc_body(x_ref, we_ref, be_ref, wl_ref, h_ref, *p_refs):
    h = jnp.maximum(
        jnp.dot(x_ref[...], we_ref[...],
                preferred_element_type=jnp.float32) + be_ref[...], 0.0)
    p = jnp.dot(h, wl_ref[...], preferred_element_type=jnp.float32)
    h_ref[...] = h
    _split_q(p, p_refs)


def _enc_call(x, W_enc, b_enc2, Wl0):
    qspecs, qshapes = _q_outs()
    return pl.pallas_call(
        _enc_body,
        grid=(GRID,),
        in_specs=[_row_spec(D), _full_spec((D, D)), _full_spec((1, D)),
                  _full_spec((D, D))],
        out_specs=[_row_spec(D)] + qspecs,
        out_shape=[jax.ShapeDtypeStruct((N, D), jnp.float32)] + qshapes,
    )(x, W_enc, b_enc2, Wl0)


def _post_common(a_refs, cnt_ref, h_ref, wr_ref, bl_ref, g_ref, be_ref):
    cnt = jnp.maximum(cnt_ref[...], 1.0)
    mean_wl = jnp.concatenate([a[...] for a in a_refs], axis=1) / cnt
    h = h_ref[...]
    u = mean_wl + bl_ref[...] + jnp.dot(
        h, wr_ref[...], preferred_element_type=jnp.float32) + h
    r = jnp.maximum(u, 0.0)
    mu = jnp.mean(r, axis=1, keepdims=True)
    var = jnp.mean((r - mu) ** 2, axis=1, keepdims=True)
    return (r - mu) / jnp.sqrt(var + 1e-5) * g_ref[...] + be_ref[...]


def _mid_body(a0, a1, a2, a3, cnt_ref, h_ref, wr_ref, bl_ref, g_ref,
              be_ref, wln_ref, hn_ref, *p_refs):
    hn = _post_common((a0, a1, a2, a3), cnt_ref, h_ref, wr_ref, bl_ref,
                      g_ref, be_ref)
    hn_ref[...] = hn
    p = jnp.dot(hn, wln_ref[...], preferred_element_type=jnp.float32)
    _split_q(p, p_refs)


def _mid_call(aq, cnt2d, h, Wr, bl2, g2, be2, Wl_next):
    qspecs, qshapes = _q_outs()
    return pl.pallas_call(
        _mid_body,
        grid=(GRID,),
        in_specs=[_row_spec(DQ)] * 4 + [_row_spec(1), _row_spec(D),
                  _full_spec((D, D)), _full_spec((1, D)), _full_spec((1, D)),
                  _full_spec((1, D)), _full_spec((D, D))],
        out_specs=[_row_spec(D)] + qspecs,
        out_shape=[jax.ShapeDtypeStruct((N, D), jnp.float32)] + qshapes,
    )(*aq, cnt2d, h, Wr, bl2, g2, be2, Wl_next)


def _last_body(a0, a1, a2, a3, cnt_ref, h_ref, henc_ref, h1_ref, wr_ref,
               bl_ref, g_ref, be_ref, out_ref):
    hn = _post_common((a0, a1, a2, a3), cnt_ref, h_ref, wr_ref, bl_ref,
                      g_ref, be_ref)
    out_ref[...] = hn + henc_ref[...] + h1_ref[...]


def _last_call(aq, cnt2d, h, h_enc, h1, Wr, bl2, g2, be2):
    return pl.pallas_call(
        _last_body,
        grid=(GRID,),
        in_specs=[_row_spec(DQ)] * 4 + [_row_spec(1), _row_spec(D),
                  _row_spec(D), _row_spec(D), _full_spec((D, D)),
                  _full_spec((1, D)), _full_spec((1, D)), _full_spec((1, D))],
        out_specs=_row_spec(D),
        out_shape=jax.ShapeDtypeStruct((N, D), jnp.float32),
    )(*aq, cnt2d, h, h_enc, h1, Wr, bl2, g2, be2)


def kernel(x, edge_index, W_enc, b_enc,
           Wl0, bl0, Wr0, g0, be0,
           Wl1, bl1, Wr1, g1, be1,
           Wl2, bl2, Wr2, g2, be2):
    pad = E_PAD - E
    src2d = jnp.concatenate(
        [edge_index[0], jnp.zeros((pad,), jnp.int32)]).reshape(E_PAD // C, C)
    dst2d = jnp.concatenate(
        [edge_index[1],
         jnp.full((pad,), DUMP, jnp.int32)]).reshape(E_PAD // C, C)
    zrows = jnp.zeros((N_TILE, DQ), jnp.float32)
    zcnt = jnp.zeros((N2,), jnp.float32)
    ones = jnp.ones((C,), jnp.float32)

    r1 = lambda v: v.reshape(1, D)

    enc = _enc_call(x, W_enc, r1(b_enc), Wl0)
    h_enc, pq = enc[0], enc[1:]
    sc = _sc_agg(pq, src2d, dst2d, zrows, zcnt, ones, True)
    aq, cntp = sc[:4], sc[4]
    cnt2d = cntp[:N].reshape(N, 1)
    mid = _mid_call(aq, cnt2d, h_enc, Wr0, r1(bl0), r1(g0), r1(be0), Wl1)
    h1, qq = mid[0], mid[1:]
    bq = _sc_agg(qq, src2d, dst2d, zrows, zcnt, ones, False)
    mid = _mid_call(bq, cnt2d, h1, Wr1, r1(bl1), r1(g1), r1(be1), Wl2)
    h2, tq = mid[0], mid[1:]
    cq = _sc_agg(tq, src2d, dst2d, zrows, zcnt, ones, False)
    out = _last_call(cq, cnt2d, h2, h_enc, h1, Wr2, r1(bl2), r1(g2),
                     r1(be2))
    return out
